# baseline (speedup 1.0000x reference)
import functools

import jax
import jax.numpy as jnp
from jax.experimental import pallas as pl
from jax.experimental.pallas import tpu as pltpu

NUM_EXPERTS = 16
TOP_K = 2
RANK = 64
LOG2_RANK = 6


def _moe_block_kernel(x_ref, wr_ref, a_ref, b_ref, scale_ref, out_ref):
    x = x_ref[...]
    logits = jnp.dot(x, wr_ref[...],
                     preferred_element_type=jnp.float32)
    eids = jax.lax.broadcasted_iota(jnp.int32, logits.shape, 1)
    m1 = jnp.max(logits, axis=-1, keepdims=True)
    i1 = jnp.min(jnp.where(logits == m1, eids, NUM_EXPERTS),
                 axis=-1, keepdims=True)
    masked = jnp.where(eids == i1, -jnp.inf, logits)
    m2 = jnp.max(masked, axis=-1, keepdims=True)
    i2 = jnp.min(jnp.where(masked == m2, eids, NUM_EXPERTS),
                 axis=-1, keepdims=True)
    g1 = 1.0 / (1.0 + jnp.exp(m2 - m1))
    g2 = 1.0 - g1
    h = jnp.dot(x, a_ref[...],
                preferred_element_type=jnp.float32)
    lane_e = jax.lax.broadcasted_iota(jnp.int32, h.shape, 1) >> LOG2_RANK
    w_exp = (jnp.where(lane_e == i1, g1, 0.0)
             + jnp.where(lane_e == i2, g2, 0.0)) * scale_ref[...][None, :]
    hg = (h * w_exp).astype(jnp.bfloat16)
    out_ref[...] = jnp.dot(hg, b_ref[...],
                           preferred_element_type=jnp.float32)


@functools.partial(jax.jit, static_argnames=())
def kernel(x, Wr, A, B, scale):
    T, IN = x.shape
    E = Wr.shape[1]
    OUT = B.shape[2]
    x16 = x.astype(jnp.bfloat16)
    Wr16 = Wr.astype(jnp.bfloat16)
    A_cat = A.transpose(1, 0, 2).reshape(IN, E * RANK).astype(jnp.bfloat16)
    B_cat = B.reshape(E * RANK, OUT).astype(jnp.bfloat16)
    scale_exp = jnp.repeat(scale, RANK)
    BT = 512
    grid = (T // BT,)
    return pl.pallas_call(
        _moe_block_kernel,
        grid=grid,
        in_specs=[
            pl.BlockSpec((BT, IN), lambda i: (i, 0)),
            pl.BlockSpec((IN, E), lambda i: (0, 0)),
            pl.BlockSpec((IN, E * RANK), lambda i: (0, 0)),
            pl.BlockSpec((E * RANK, OUT), lambda i: (0, 0)),
            pl.BlockSpec((E * RANK,), lambda i: (0,)),
        ],
        out_specs=pl.BlockSpec((BT, OUT), lambda i: (i, 0)),
        out_shape=jax.ShapeDtypeStruct((T, OUT), jnp.float32),
    )(x16, Wr16, A_cat, B_cat, scale_exp)


# speedup vs baseline: 1.2862x; 1.2862x over previous
import functools

import jax
import jax.numpy as jnp
from jax.experimental import pallas as pl
from jax.experimental.pallas import tpu as pltpu

NUM_EXPERTS = 16
TOP_K = 2
RANK = 64
LOG2_RANK = 6


def _moe_block_kernel(x_ref, wr_ref, a_ref, b_ref, scale_ref, out_ref):
    x = x_ref[...]
    logits = jnp.dot(x, wr_ref[...],
                     preferred_element_type=jnp.float32)
    eids = jax.lax.broadcasted_iota(jnp.int32, logits.shape, 1)
    m1 = jnp.max(logits, axis=-1, keepdims=True)
    i1 = jnp.min(jnp.where(logits == m1, eids, NUM_EXPERTS),
                 axis=-1, keepdims=True)
    masked = jnp.where(eids == i1, -jnp.inf, logits)
    m2 = jnp.max(masked, axis=-1, keepdims=True)
    i2 = jnp.min(jnp.where(masked == m2, eids, NUM_EXPERTS),
                 axis=-1, keepdims=True)
    g1 = 1.0 / (1.0 + jnp.exp(m2 - m1))
    g2 = 1.0 - g1
    h = jnp.dot(x, a_ref[...],
                preferred_element_type=jnp.float32)
    lane_e = jax.lax.broadcasted_iota(jnp.int32, h.shape, 1) >> LOG2_RANK
    w_exp = (jnp.where(lane_e == i1, g1, 0.0)
             + jnp.where(lane_e == i2, g2, 0.0)) * scale_ref[...][None, :]
    hg = (h * w_exp).astype(jnp.bfloat16)
    out_ref[...] = jnp.dot(hg, b_ref[...],
                           preferred_element_type=jnp.float32)


@functools.partial(jax.jit, static_argnames=())
def kernel(x, Wr, A, B, scale):
    T, IN = x.shape
    E = Wr.shape[1]
    OUT = B.shape[2]
    A_cat = A.transpose(1, 0, 2).reshape(IN, E * RANK).astype(jnp.bfloat16)
    B_cat = B.reshape(E * RANK, OUT).astype(jnp.bfloat16)
    scale_exp = jnp.repeat(scale, RANK)
    BT = 1024
    grid = (T // BT,)
    return pl.pallas_call(
        _moe_block_kernel,
        grid=grid,
        in_specs=[
            pl.BlockSpec((BT, IN), lambda i: (i, 0)),
            pl.BlockSpec((IN, E), lambda i: (0, 0)),
            pl.BlockSpec((IN, E * RANK), lambda i: (0, 0)),
            pl.BlockSpec((E * RANK, OUT), lambda i: (0, 0)),
            pl.BlockSpec((E * RANK,), lambda i: (0,)),
        ],
        out_specs=pl.BlockSpec((BT, OUT), lambda i: (i, 0)),
        out_shape=jax.ShapeDtypeStruct((T, OUT), jnp.float32),
    )(x, Wr, A_cat, B_cat, scale_exp)


# speedup vs baseline: 1.3838x; 1.0759x over previous
import functools

import jax
import jax.numpy as jnp
from jax.experimental import pallas as pl
from jax.experimental.pallas import tpu as pltpu

NUM_EXPERTS = 16
TOP_K = 2
RANK = 64
LOG2_RANK = 6


def _moe_block_kernel(x_ref, wr_ref, a_ref, b_ref, out_ref):
    x = x_ref[...]
    logits = jnp.dot(x, wr_ref[...],
                     preferred_element_type=jnp.float32)
    eids = jax.lax.broadcasted_iota(jnp.int32, logits.shape, 1)
    m1 = jnp.max(logits, axis=-1, keepdims=True)
    i1 = jnp.min(jnp.where(logits == m1, eids, NUM_EXPERTS),
                 axis=-1, keepdims=True)
    masked = jnp.where(eids == i1, -jnp.inf, logits)
    m2 = jnp.max(masked, axis=-1, keepdims=True)
    i2 = jnp.min(jnp.where(masked == m2, eids, NUM_EXPERTS),
                 axis=-1, keepdims=True)
    g1 = 1.0 / (1.0 + jnp.exp(m2 - m1))
    g2 = 1.0 - g1
    h = jnp.dot(x.astype(jnp.bfloat16), a_ref[...],
                preferred_element_type=jnp.float32)
    lane_e = jax.lax.broadcasted_iota(jnp.int32, h.shape, 1) >> LOG2_RANK
    w_exp = jnp.where(lane_e == i1, g1,
                      jnp.where(lane_e == i2, g2, 0.0))
    out_ref[...] = jnp.dot(h * w_exp, b_ref[...],
                           preferred_element_type=jnp.float32)


@functools.partial(jax.jit, static_argnames=())
def kernel(x, Wr, A, B, scale):
    T, IN = x.shape
    E = Wr.shape[1]
    OUT = B.shape[2]
    A_cat = (A * scale[:, None, None]).transpose(1, 0, 2).reshape(
        IN, E * RANK).astype(jnp.bfloat16)
    B_cat = B.reshape(E * RANK, OUT)
    BT = 512
    grid = (T // BT,)
    return pl.pallas_call(
        _moe_block_kernel,
        grid=grid,
        in_specs=[
            pl.BlockSpec((BT, IN), lambda i: (i, 0)),
            pl.BlockSpec((IN, E), lambda i: (0, 0)),
            pl.BlockSpec((IN, E * RANK), lambda i: (0, 0)),
            pl.BlockSpec((E * RANK, OUT), lambda i: (0, 0)),
        ],
        out_specs=pl.BlockSpec((BT, OUT), lambda i: (i, 0)),
        out_shape=jax.ShapeDtypeStruct((T, OUT), jnp.float32),
    )(x, Wr, A_cat, B_cat)
